# Initial kernel scaffold; baseline (speedup 1.0000x reference)
import jax
import jax.numpy as jnp
from jax.experimental import pallas as pl


def kernel(embedded_data, embedding_matrix):
    raise NotImplementedError("write your pallas kernel here")


# speedup vs baseline: 1.4956x; 1.4956x over previous
import jax
import jax.numpy as jnp
from jax.experimental import pallas as pl

_BLK = 2048
_OUT_W = 256
_ROWS = _BLK // _OUT_W


def _nn_kernel(e_ref, x_ref, o_ref):
    e = e_ref[...]
    x = x_ref[...]
    s = jax.lax.dot_general(
        e, x, (((1,), (0,)), ((), ())),
        preferred_element_type=jnp.float32)
    q2 = jnp.sum(x * x, axis=0, keepdims=True)
    c2 = jnp.sum(e * e, axis=1, keepdims=True)
    d2 = (q2 - 2.0 * s) + c2
    idx = jnp.argmin(d2, axis=0).astype(jnp.int32)
    o_ref[...] = idx.reshape(_ROWS, _OUT_W)


def kernel(embedded_data, embedding_matrix):
    b, d, D, H, W = embedded_data.shape
    n = D * H * W
    k = embedding_matrix.shape[0]
    x = embedded_data.reshape(d, n)
    out = pl.pallas_call(
        _nn_kernel,
        grid=(n // _BLK,),
        in_specs=[
            pl.BlockSpec((k, d), lambda i: (0, 0)),
            pl.BlockSpec((d, _BLK), lambda i: (0, i)),
        ],
        out_specs=pl.BlockSpec((_ROWS, _OUT_W), lambda i: (i, 0)),
        out_shape=jax.ShapeDtypeStruct((n // _OUT_W, _OUT_W), jnp.int32),
    )(embedding_matrix, x)
    return out.reshape(D, H, W)


# speedup vs baseline: 1.6756x; 1.1204x over previous
import jax
import jax.numpy as jnp
from jax.experimental import pallas as pl

_BLK = 2048
_OUT_W = 256
_ROWS = _BLK // _OUT_W


def _nn_kernel(e_ref, x_ref, o_ref):
    e = e_ref[...]
    x = x_ref[...]
    s = jax.lax.dot_general(
        e, x, (((1,), (0,)), ((), ())),
        preferred_element_type=jnp.float32)
    hc2 = 0.5 * jnp.sum(e * e, axis=1, keepdims=True)
    idx = jnp.argmin(hc2 - s, axis=0).astype(jnp.int32)
    o_ref[...] = idx.reshape(_ROWS, _OUT_W)


def kernel(embedded_data, embedding_matrix):
    b, d, D, H, W = embedded_data.shape
    n = D * H * W
    k = embedding_matrix.shape[0]
    x = embedded_data.reshape(d, n)
    out = pl.pallas_call(
        _nn_kernel,
        grid=(n // _BLK,),
        in_specs=[
            pl.BlockSpec((k, d), lambda i: (0, 0)),
            pl.BlockSpec((d, _BLK), lambda i: (0, i)),
        ],
        out_specs=pl.BlockSpec((_ROWS, _OUT_W), lambda i: (i, 0)),
        out_shape=jax.ShapeDtypeStruct((n // _OUT_W, _OUT_W), jnp.int32),
    )(embedding_matrix, x)
    return out.reshape(D, H, W)


# speedup vs baseline: 1.9118x; 1.1410x over previous
import jax
import jax.numpy as jnp
from jax.experimental import pallas as pl

_BLK = 4096
_OUT_W = 256
_ROWS = _BLK // _OUT_W


def _nn_kernel(e_ref, x_ref, o_ref):
    e = e_ref[...]
    x = x_ref[...]
    s = jax.lax.dot_general(
        e, x, (((1,), (0,)), ((), ())),
        preferred_element_type=jnp.float32)
    hc2 = 0.5 * jnp.sum(e * e, axis=1, keepdims=True)
    idx = jnp.argmin(hc2 - s, axis=0).astype(jnp.int32)
    o_ref[...] = idx.reshape(_ROWS, _OUT_W)


def kernel(embedded_data, embedding_matrix):
    b, d, D, H, W = embedded_data.shape
    n = D * H * W
    k = embedding_matrix.shape[0]
    x = embedded_data.reshape(d, n)
    out = pl.pallas_call(
        _nn_kernel,
        grid=(n // _BLK,),
        in_specs=[
            pl.BlockSpec((k, d), lambda i: (0, 0)),
            pl.BlockSpec((d, _BLK), lambda i: (0, i)),
        ],
        out_specs=pl.BlockSpec((_ROWS, _OUT_W), lambda i: (i, 0)),
        out_shape=jax.ShapeDtypeStruct((n // _OUT_W, _OUT_W), jnp.int32),
    )(embedding_matrix, x)
    return out.reshape(D, H, W)


# speedup vs baseline: 1.9594x; 1.0249x over previous
import jax
import jax.numpy as jnp
from jax.experimental import pallas as pl

_BLK = 8192
_OUT_W = 256
_ROWS = _BLK // _OUT_W


def _nn_kernel(e_ref, x_ref, o_ref):
    e = e_ref[...]
    x = x_ref[...]
    s = jax.lax.dot_general(
        e, x, (((1,), (0,)), ((), ())),
        preferred_element_type=jnp.float32)
    hc2 = 0.5 * jnp.sum(e * e, axis=1, keepdims=True)
    idx = jnp.argmin(hc2 - s, axis=0).astype(jnp.int32)
    o_ref[...] = idx.reshape(_ROWS, _OUT_W)


def kernel(embedded_data, embedding_matrix):
    b, d, D, H, W = embedded_data.shape
    n = D * H * W
    k = embedding_matrix.shape[0]
    x = embedded_data.reshape(d, n)
    out = pl.pallas_call(
        _nn_kernel,
        grid=(n // _BLK,),
        in_specs=[
            pl.BlockSpec((k, d), lambda i: (0, 0)),
            pl.BlockSpec((d, _BLK), lambda i: (0, i)),
        ],
        out_specs=pl.BlockSpec((_ROWS, _OUT_W), lambda i: (i, 0)),
        out_shape=jax.ShapeDtypeStruct((n // _OUT_W, _OUT_W), jnp.int32),
    )(embedding_matrix, x)
    return out.reshape(D, H, W)


# speedup vs baseline: 1.9816x; 1.0113x over previous
import jax
import jax.numpy as jnp
from jax.experimental import pallas as pl

_BLK = 16384
_OUT_W = 256
_ROWS = _BLK // _OUT_W


def _nn_kernel(e_ref, x_ref, o_ref):
    e = e_ref[...]
    x = x_ref[...]
    s = jax.lax.dot_general(
        e, x, (((1,), (0,)), ((), ())),
        preferred_element_type=jnp.float32)
    hc2 = 0.5 * jnp.sum(e * e, axis=1, keepdims=True)
    idx = jnp.argmin(hc2 - s, axis=0).astype(jnp.int32)
    o_ref[...] = idx.reshape(_ROWS, _OUT_W)


def kernel(embedded_data, embedding_matrix):
    b, d, D, H, W = embedded_data.shape
    n = D * H * W
    k = embedding_matrix.shape[0]
    x = embedded_data.reshape(d, n)
    out = pl.pallas_call(
        _nn_kernel,
        grid=(n // _BLK,),
        in_specs=[
            pl.BlockSpec((k, d), lambda i: (0, 0)),
            pl.BlockSpec((d, _BLK), lambda i: (0, i)),
        ],
        out_specs=pl.BlockSpec((_ROWS, _OUT_W), lambda i: (i, 0)),
        out_shape=jax.ShapeDtypeStruct((n // _OUT_W, _OUT_W), jnp.int32),
    )(embedding_matrix, x)
    return out.reshape(D, H, W)
